# baseline (speedup 1.0000x reference)
import functools

import jax
import jax.numpy as jnp
import numpy as np
from jax.experimental import pallas as pl
from jax.experimental.pallas import tpu as pltpu

NUM_EXPERTS = 64
TOP_K = 8
TOKENS = 4 * 4096
BLOCK = 2048
GRID = TOKENS // BLOCK


def _router_body(g_ref, w_ref, i_ref, loss_ref, psum_ref):
    step = pl.program_id(0)
    vals = g_ref[...]
    orig = vals
    eidx = jax.lax.broadcasted_iota(jnp.int32, (BLOCK, NUM_EXPERTS), 1)

    ws = []
    ids = []
    for _ in range(TOP_K):
        m = jnp.max(vals, axis=-1, keepdims=True)
        cand = jnp.where(vals == m, eidx, NUM_EXPERTS)
        am = jnp.min(cand, axis=-1, keepdims=True)
        ws.append(m)
        ids.append(am)
        vals = jnp.where(eidx == am, -jnp.inf, vals)

    w = jnp.concatenate(ws, axis=-1)
    idx = jnp.concatenate(ids, axis=-1)
    w = w / jnp.sum(w, axis=-1, keepdims=True)
    w_ref[...] = w
    i_ref[...] = idx

    part = jnp.sum(orig, axis=0, keepdims=True)

    @pl.when(step == 0)
    def _():
        psum_ref[...] = part

    @pl.when(step > 0)
    def _():
        psum_ref[...] = psum_ref[...] + part

    @pl.when(step == GRID - 1)
    def _():
        gate_mean = psum_ref[...] * (1.0 / TOKENS)
        entropy = -jnp.sum(gate_mean * jnp.log(gate_mean + 1e-08))
        loss = 1.0 - entropy / np.log(NUM_EXPERTS).astype(np.float32)
        loss_ref[...] = jnp.reshape(loss, (1, 1))


@jax.jit
def kernel(gate_weights):
    b, s, e = gate_weights.shape
    g = gate_weights.reshape(TOKENS, NUM_EXPERTS)
    w, idx, loss = pl.pallas_call(
        _router_body,
        grid=(GRID,),
        in_specs=[pl.BlockSpec((BLOCK, NUM_EXPERTS), lambda i: (i, 0))],
        out_specs=[
            pl.BlockSpec((BLOCK, TOP_K), lambda i: (i, 0)),
            pl.BlockSpec((BLOCK, TOP_K), lambda i: (i, 0)),
            pl.BlockSpec((1, 1), lambda i: (0, 0)),
        ],
        out_shape=[
            jax.ShapeDtypeStruct((TOKENS, TOP_K), jnp.float32),
            jax.ShapeDtypeStruct((TOKENS, TOP_K), jnp.int32),
            jax.ShapeDtypeStruct((1, 1), jnp.float32),
        ],
        scratch_shapes=[pltpu.VMEM((1, NUM_EXPERTS), jnp.float32)],
    )(g)
    return (
        w.reshape(b, s, TOP_K),
        idx.reshape(b, s, TOP_K),
        loss.reshape(()),
    )
